# Initial kernel scaffold; baseline (speedup 1.0000x reference)
import jax
import jax.numpy as jnp
from jax.experimental import pallas as pl


def kernel(node_inp, node_type, edge_index, edge_type, node_position, Kw, Kb, Qw, Qb, Vw, Vb, Aw, Ab, relation_pri, relation_att, relation_msg, relation_s2u, skip):
    raise NotImplementedError("write your pallas kernel here")


# speedup vs baseline: 7.5166x; 7.5166x over previous
import functools
import math

import jax
import jax.numpy as jnp
from jax import lax
from jax.experimental import pallas as pl

N = 10000
E = 320000
DIM = 128
T = 3
R = 4
NH = 8
DK = 16

BLK = 400


def _prep_body(x_ref, nt_ref, Kw_ref, Kb_ref, Qw_ref, Qb_ref, Vw_ref, Vb_ref,
               BDatt_ref, BDmsg_ref, pri_ref,
               qhat_ref, kA3_ref, vM_ref):
    x = x_ref[...]
    nt2 = nt_ref[0]

    def typed_proj(w_ref, b_ref):
        b = b_ref[...]
        outs = [jnp.dot(x, w_ref[t], preferred_element_type=jnp.float32)
                + b[t:t + 1, :] for t in range(T)]
        sel = outs[T - 1]
        for t in range(T - 2, -1, -1):
            sel = jnp.where(nt2 == t, outs[t], sel)
        return sel

    k_node = typed_proj(Kw_ref, Kb_ref)
    q_node = typed_proj(Qw_ref, Qb_ref)
    v_node = typed_proj(Vw_ref, Vb_ref)
    qhat_ref[...] = q_node * (1.0 / math.sqrt(DK))
    for r in range(R):
        kA_r = jnp.dot(k_node, BDatt_ref[r], preferred_element_type=jnp.float32)
        vM_ref[r] = jnp.dot(v_node, BDmsg_ref[r], preferred_element_type=jnp.float32)
        for ti in range(T):
            pri = pri_ref[ti, r]
            pri_sel = pri[T - 1:T, :]
            for tt in range(T - 2, -1, -1):
                pri_sel = jnp.where(nt2 == tt, pri[tt:tt + 1, :], pri_sel)
            kA3_ref[ti, r] = kA_r * pri_sel


def _run_prep(node_inp, nt3, Kw, Kb, Qw, Qb, Vw, Vb, BDatt, BDmsg, pri16):
    nb = N // BLK
    grid = (nb,)
    full = lambda *s: pl.BlockSpec(s, lambda ib: tuple(0 for _ in s))
    in_specs = [
        pl.BlockSpec((BLK, DIM), lambda ib: (ib, 0)),
        pl.BlockSpec((1, BLK, 1), lambda ib: (ib, 0, 0)),
        full(T, DIM, DIM), full(T, DIM),
        full(T, DIM, DIM), full(T, DIM),
        full(T, DIM, DIM), full(T, DIM),
        full(R, DIM, DIM), full(R, DIM, DIM),
        full(T, R, T, DIM),
    ]
    out_specs = [
        pl.BlockSpec((BLK, DIM), lambda ib: (ib, 0)),
        pl.BlockSpec((T, R, BLK, DIM), lambda ib: (0, 0, ib, 0)),
        pl.BlockSpec((R, BLK, DIM), lambda ib: (0, ib, 0)),
    ]
    out_shape = [
        jax.ShapeDtypeStruct((N, DIM), jnp.float32),
        jax.ShapeDtypeStruct((T, R, N, DIM), jnp.float32),
        jax.ShapeDtypeStruct((R, N, DIM), jnp.float32),
    ]
    return pl.pallas_call(
        _prep_body, grid=grid, in_specs=in_specs, out_specs=out_specs,
        out_shape=out_shape,
    )(node_inp, nt3, Kw, Kb, Qw, Qb, Vw, Vb, BDatt, BDmsg, pri16)


def _final_body(a_ref, dexp_ref, xw_ref, wm_ref, nt_ref, x_ref,
                Wsp_ref, bsp_ref, Aw_ref, Ab_ref, alpha_ref, out_ref):
    a = a_ref[0] + a_ref[1]
    a = a / dexp_ref[...]
    sp = jnp.dot(xw_ref[...], Wsp_ref[...], preferred_element_type=jnp.float32)
    sp = sp + bsp_ref[...]
    wm = wm_ref[0]
    nt2 = nt_ref[0]
    a = a + jnp.where(wm > 0, sp, 0.0)
    g = 0.5 * a * (1.0 + lax.erf(a * (1.0 / math.sqrt(2.0))))
    Abv = Ab_ref[...]
    outs = [jnp.dot(g, Aw_ref[t], preferred_element_type=jnp.float32)
            + Abv[t:t + 1, :] for t in range(T)]
    sel = outs[T - 1]
    alv = alpha_ref[...]
    al = alv[T - 1:T, :]
    for t in range(T - 2, -1, -1):
        m = nt2 == t
        sel = jnp.where(m, outs[t], sel)
        al = jnp.where(m, alv[t:t + 1, :], al)
    out_ref[...] = sel * al + x_ref[...] * (1.0 - al)


def _run_final(aggr2, dexp, xw, wm3, nt3, node_inp, Wsp, bsp, Aw, Ab, alpha8):
    nb = N // BLK
    full = lambda *s: pl.BlockSpec(s, lambda ib: tuple(0 for _ in s))
    in_specs = [
        pl.BlockSpec((2, BLK, DIM), lambda ib: (0, ib, 0)),
        pl.BlockSpec((BLK, DIM), lambda ib: (ib, 0)),
        pl.BlockSpec((BLK, DIM), lambda ib: (ib, 0)),
        pl.BlockSpec((1, BLK, 1), lambda ib: (ib, 0, 0)),
        pl.BlockSpec((1, BLK, 1), lambda ib: (ib, 0, 0)),
        pl.BlockSpec((BLK, DIM), lambda ib: (ib, 0)),
        full(DIM, DIM), full(1, DIM),
        full(T, DIM, DIM), full(T, DIM),
        full(T, DIM),
    ]
    return pl.pallas_call(
        _final_body, grid=(nb,), in_specs=in_specs,
        out_specs=pl.BlockSpec((BLK, DIM), lambda ib: (ib, 0)),
        out_shape=jax.ShapeDtypeStruct((N, DIM), jnp.float32),
    )(aggr2, dexp, xw, wm3, nt3, node_inp, Wsp, bsp, Aw, Ab, alpha8)


def _edge_phase(qhat, kA3f, vMf, nt, i, j, r):
    ti = nt[i]
    idxk = (ti * R + r) * N + j
    idxv = r * N + j
    qg = qhat[i].reshape(E, NH, DK)
    kg = kA3f[idxk].reshape(E, NH, DK)
    s = (qg * kg).sum(-1)
    w = jnp.exp(s)
    denom = jax.ops.segment_sum(w, i, num_segments=N)
    vg = vMf[idxv].reshape(E, NH, DK)
    aggr = jax.ops.segment_sum((vg * w[:, :, None]).reshape(E, DIM), i,
                               num_segments=N)
    aggr2 = jnp.stack([aggr, jnp.zeros_like(aggr)])
    return aggr2, denom


def kernel(node_inp, node_type, edge_index, edge_type, node_position,
           Kw, Kb, Qw, Qb, Vw, Vb, Aw, Ab,
           relation_pri, relation_att, relation_msg, relation_s2u, skip):
    nt3 = node_type.reshape(N // BLK, BLK, 1)
    def blockdiag2(rel):
        out = jnp.zeros((R, DIM, DIM), jnp.float32)
        for h in range(NH):
            out = out.at[:, h * DK:(h + 1) * DK, h * DK:(h + 1) * DK].set(rel[:, h])
        return out
    BDatt = blockdiag2(relation_att)
    BDmsg = blockdiag2(relation_msg)
    pri16 = jnp.repeat(relation_pri, DK, axis=-1)

    qhat, kA3, vM = _run_prep(node_inp, nt3, Kw, Kb, Qw, Qb, Vw, Vb,
                              BDatt, BDmsg, pri16)
    kA3f = kA3.reshape(T * R * N, DIM)
    vMf = vM.reshape(R * N, DIM)

    j = edge_index[0]
    i = edge_index[1]
    aggr2, denom = _edge_phase(qhat, kA3f, vMf, node_type, i, j, edge_type)

    dexp = jnp.repeat(denom + 1e-16, DK, axis=-1)

    mask = edge_type == 0
    idx_j = jnp.where(mask, j, N)
    win = jnp.full((N,), -1, jnp.int32).at[idx_j].set(
        jnp.arange(E, dtype=jnp.int32), mode='drop')
    iw = i[jnp.clip(win, 0, E - 1)]
    xw = node_inp[iw]
    wm3 = (win >= 0).astype(jnp.float32).reshape(N // BLK, BLK, 1)

    Wsp = Vw[1] @ relation_s2u[0]
    bsp = (Vb[1] @ relation_s2u[0]).reshape(1, DIM)
    alpha8 = jnp.broadcast_to(jax.nn.sigmoid(skip)[:, None], (T, DIM))

    return _run_final(aggr2, dexp, xw, wm3, nt3, node_inp,
                      Wsp, bsp, Aw, Ab, alpha8)


# speedup vs baseline: 12.2835x; 1.6342x over previous
import functools
import math

import jax
import jax.numpy as jnp
from jax import lax
from jax.experimental import pallas as pl
from jax.experimental.pallas import tpu as pltpu
from jax.experimental.pallas import tpu_sc as plsc

N = 10000
E = 320000
DIM = 128
T = 3
R = 4
NH = 8
DK = 16

BLK = 400


def _prep_body(x_ref, nt_ref, Kw_ref, Kb_ref, Qw_ref, Qb_ref, Vw_ref, Vb_ref,
               BDatt_ref, BDmsg_ref, pri_ref,
               qhat_ref, kA3_ref, vM_ref):
    x = x_ref[...]
    nt2 = nt_ref[0]

    def typed_proj(w_ref, b_ref):
        b = b_ref[...]
        outs = [jnp.dot(x, w_ref[t], preferred_element_type=jnp.float32)
                + b[t:t + 1, :] for t in range(T)]
        sel = outs[T - 1]
        for t in range(T - 2, -1, -1):
            sel = jnp.where(nt2 == t, outs[t], sel)
        return sel

    k_node = typed_proj(Kw_ref, Kb_ref)
    q_node = typed_proj(Qw_ref, Qb_ref)
    v_node = typed_proj(Vw_ref, Vb_ref)
    qhat_ref[...] = q_node * (1.0 / math.sqrt(DK))
    for r in range(R):
        kA_r = jnp.dot(k_node, BDatt_ref[r], preferred_element_type=jnp.float32)
        vM_ref[r] = jnp.dot(v_node, BDmsg_ref[r], preferred_element_type=jnp.float32)
        for ti in range(T):
            pri = pri_ref[ti, r]
            pri_sel = pri[T - 1:T, :]
            for tt in range(T - 2, -1, -1):
                pri_sel = jnp.where(nt2 == tt, pri[tt:tt + 1, :], pri_sel)
            kA3_ref[ti, r] = kA_r * pri_sel


def _run_prep(node_inp, nt3, Kw, Kb, Qw, Qb, Vw, Vb, BDatt, BDmsg, pri16):
    nb = N // BLK
    grid = (nb,)
    full = lambda *s: pl.BlockSpec(s, lambda ib: tuple(0 for _ in s))
    in_specs = [
        pl.BlockSpec((BLK, DIM), lambda ib: (ib, 0)),
        pl.BlockSpec((1, BLK, 1), lambda ib: (ib, 0, 0)),
        full(T, DIM, DIM), full(T, DIM),
        full(T, DIM, DIM), full(T, DIM),
        full(T, DIM, DIM), full(T, DIM),
        full(R, DIM, DIM), full(R, DIM, DIM),
        full(T, R, T, DIM),
    ]
    out_specs = [
        pl.BlockSpec((BLK, DIM), lambda ib: (ib, 0)),
        pl.BlockSpec((T, R, BLK, DIM), lambda ib: (0, 0, ib, 0)),
        pl.BlockSpec((R, BLK, DIM), lambda ib: (0, ib, 0)),
    ]
    out_shape = [
        jax.ShapeDtypeStruct((N, DIM), jnp.float32),
        jax.ShapeDtypeStruct((T, R, N, DIM), jnp.float32),
        jax.ShapeDtypeStruct((R, N, DIM), jnp.float32),
    ]
    return pl.pallas_call(
        _prep_body, grid=grid, in_specs=in_specs, out_specs=out_specs,
        out_shape=out_shape,
    )(node_inp, nt3, Kw, Kb, Qw, Qb, Vw, Vb, BDatt, BDmsg, pri16)


def _final_body(a_ref, dexp_ref, xw_ref, wm_ref, nt_ref, x_ref,
                Wsp_ref, bsp_ref, Aw_ref, Ab_ref, alpha_ref, out_ref):
    a = a_ref[0] + a_ref[1]
    a = a / dexp_ref[...]
    sp = jnp.dot(xw_ref[...], Wsp_ref[...], preferred_element_type=jnp.float32)
    sp = sp + bsp_ref[...]
    wm = wm_ref[0]
    nt2 = nt_ref[0]
    a = a + jnp.where(wm > 0, sp, 0.0)
    g = 0.5 * a * (1.0 + lax.erf(a * (1.0 / math.sqrt(2.0))))
    Abv = Ab_ref[...]
    outs = [jnp.dot(g, Aw_ref[t], preferred_element_type=jnp.float32)
            + Abv[t:t + 1, :] for t in range(T)]
    sel = outs[T - 1]
    alv = alpha_ref[...]
    al = alv[T - 1:T, :]
    for t in range(T - 2, -1, -1):
        m = nt2 == t
        sel = jnp.where(m, outs[t], sel)
        al = jnp.where(m, alv[t:t + 1, :], al)
    out_ref[...] = sel * al + x_ref[...] * (1.0 - al)


def _run_final(aggr2, dexp, xw, wm3, nt3, node_inp, Wsp, bsp, Aw, Ab, alpha8):
    nb = N // BLK
    full = lambda *s: pl.BlockSpec(s, lambda ib: tuple(0 for _ in s))
    in_specs = [
        pl.BlockSpec((2, BLK, DIM), lambda ib: (0, ib, 0)),
        pl.BlockSpec((BLK, DIM), lambda ib: (ib, 0)),
        pl.BlockSpec((BLK, DIM), lambda ib: (ib, 0)),
        pl.BlockSpec((1, BLK, 1), lambda ib: (ib, 0, 0)),
        pl.BlockSpec((1, BLK, 1), lambda ib: (ib, 0, 0)),
        pl.BlockSpec((BLK, DIM), lambda ib: (ib, 0)),
        full(DIM, DIM), full(1, DIM),
        full(T, DIM, DIM), full(T, DIM),
        full(T, DIM),
    ]
    return pl.pallas_call(
        _final_body, grid=(nb,), in_specs=in_specs,
        out_specs=pl.BlockSpec((BLK, DIM), lambda ib: (ib, 0)),
        out_shape=jax.ShapeDtypeStruct((N, DIM), jnp.float32),
    )(aggr2, dexp, xw, wm3, nt3, node_inp, Wsp, bsp, Aw, Ab, alpha8)


NC = 2
NS = 16
NW = NC * NS
EPW = E // NW
CH = 80
NCHUNK = EPW // CH
NACC = 10240
NPT = NACC // NS


def _edge_sc_body(qhat_hbm, kA3_hbm, vM_hbm, nt_hbm, dst_hbm, src_hbm, et_hbm,
                  aggr_out, den_out,
                  iv, jv, rv, tiv, idxk, idxv, qrows, krows, vrows, wrows,
                  ash, dsh, sem0, sem1, sem2):
    cid = lax.axis_index("c")
    sid = lax.axis_index("s")
    wid = cid * NS + sid
    f16z = jnp.zeros((16,), jnp.float32)

    def _zrow(rr, carry):
        for cc in range(DIM // 16):
            qrows[rr, pl.ds(cc * 16, 16)] = f16z
        wrows[rr, :] = f16z
        return carry
    lax.fori_loop(0, CH, _zrow, 0)
    for kk in range(NPT // CH):
        sl = pl.ds(sid * NPT + kk * CH, CH)
        pltpu.sync_copy(qrows, ash.at[sl])
        pltpu.sync_copy(wrows, dsh.at[sl])
    plsc.subcore_barrier()

    iota16 = jax.lax.iota(jnp.int32, 16)

    def _chunk(s, carry):
        base = wid * EPW + s * CH
        pltpu.sync_copy(dst_hbm.at[pl.ds(base, CH)], iv)
        pltpu.sync_copy(src_hbm.at[pl.ds(base, CH)], jv)
        pltpu.sync_copy(et_hbm.at[pl.ds(base, CH)], rv)
        pltpu.async_copy(nt_hbm.at[iv], tiv, sem0).wait()
        for g in range(CH // 16):
            j16 = jv[pl.ds(g * 16, 16)]
            r16 = rv[pl.ds(g * 16, 16)]
            ti16 = tiv[pl.ds(g * 16, 16)]
            idxk[pl.ds(g * 16, 16)] = (ti16 * R + r16) * N + j16
            idxv[pl.ds(g * 16, 16)] = r16 * N + j16
        cp0 = pltpu.async_copy(qhat_hbm.at[iv], qrows, sem0)
        cp1 = pltpu.async_copy(kA3_hbm.at[idxk], krows, sem1)
        cp2 = pltpu.async_copy(vM_hbm.at[idxv], vrows, sem2)
        cp0.wait()
        cp1.wait()
        for g in range(CH // 16):
            row16 = iota16 + (g * 16)

            def _dotstep(c, accs):
                new = []
                for h in range(NH):
                    col = jnp.full((16,), h * DK, jnp.int32) + c
                    qv = plsc.load_gather(qrows, [row16, col])
                    kv = plsc.load_gather(krows, [row16, col])
                    new.append(accs[h] + qv * kv)
                return tuple(new)
            accs = lax.fori_loop(0, DK, _dotstep, tuple(f16z for _ in range(NH)))
            for h in range(NH):
                wv = jnp.exp(accs[h])
                plsc.store_scatter(wrows, [row16, jnp.full((16,), h, jnp.int32)],
                                   wv)
        cp2.wait()
        def _scale(e, carry):
            for h in range(NH):
                wb = plsc.load_gather(
                    wrows, [jnp.full((16,), e, jnp.int32),
                            jnp.full((16,), h, jnp.int32)])
                vrows[e, pl.ds(h * DK, DK)] = vrows[e, pl.ds(h * DK, DK)] * wb
            return carry
        lax.fori_loop(0, CH, _scale, 0)
        pltpu.sync_copy(vrows, ash.at[iv], add=True)
        pltpu.sync_copy(wrows, dsh.at[iv], add=True)
        return carry

    lax.fori_loop(0, NCHUNK, _chunk, 0)

    plsc.subcore_barrier()
    for kk in range(5):
        sl = pl.ds(sid * NPT + kk * (NPT // 5), NPT // 5)
        pltpu.sync_copy(ash.at[sl], aggr_out.at[cid, sl])
    dslice = pl.ds(sid * NPT, NPT)
    pltpu.sync_copy(dsh.at[dslice], den_out.at[cid, dslice])


def _edge_phase(qhat, kA3f, vMf, nt, dst, src, et):
    mesh = plsc.VectorSubcoreMesh(core_axis_name="c", subcore_axis_name="s")
    run = pl.kernel(
        _edge_sc_body, mesh=mesh,
        compiler_params=pltpu.CompilerParams(needs_layout_passes=False,
                                             use_tc_tiling_on_sc=False),
        out_type=[
            jax.ShapeDtypeStruct((NC, NACC, DIM), jnp.float32),
            jax.ShapeDtypeStruct((NC, NACC, 16), jnp.float32),
        ],
        scratch_types=[
            pltpu.VMEM((CH,), jnp.int32),
            pltpu.VMEM((CH,), jnp.int32),
            pltpu.VMEM((CH,), jnp.int32),
            pltpu.VMEM((CH,), jnp.int32),
            pltpu.VMEM((CH,), jnp.int32),
            pltpu.VMEM((CH,), jnp.int32),
            pltpu.VMEM((CH, DIM), jnp.float32),
            pltpu.VMEM((CH, DIM), jnp.float32),
            pltpu.VMEM((CH, DIM), jnp.float32),
            pltpu.VMEM((CH, 16), jnp.float32),
            pltpu.VMEM_SHARED((NACC, DIM), jnp.float32),
            pltpu.VMEM_SHARED((NACC, 16), jnp.float32),
            pltpu.SemaphoreType.DMA,
            pltpu.SemaphoreType.DMA,
            pltpu.SemaphoreType.DMA,
        ],
    )
    aggr2, den2 = run(qhat, kA3f, vMf, nt, dst, src, et)
    return aggr2[:, :N], den2[:, :N, :NH].sum(0)


def kernel(node_inp, node_type, edge_index, edge_type, node_position,
           Kw, Kb, Qw, Qb, Vw, Vb, Aw, Ab,
           relation_pri, relation_att, relation_msg, relation_s2u, skip):
    nt3 = node_type.reshape(N // BLK, BLK, 1)
    def blockdiag2(rel):
        out = jnp.zeros((R, DIM, DIM), jnp.float32)
        for h in range(NH):
            out = out.at[:, h * DK:(h + 1) * DK, h * DK:(h + 1) * DK].set(rel[:, h])
        return out
    BDatt = blockdiag2(relation_att)
    BDmsg = blockdiag2(relation_msg)
    pri16 = jnp.repeat(relation_pri, DK, axis=-1)

    qhat, kA3, vM = _run_prep(node_inp, nt3, Kw, Kb, Qw, Qb, Vw, Vb,
                              BDatt, BDmsg, pri16)
    kA3f = kA3.reshape(T * R * N, DIM)
    vMf = vM.reshape(R * N, DIM)

    j = edge_index[0]
    i = edge_index[1]
    aggr2, denom = _edge_phase(qhat, kA3f, vMf, node_type, i, j, edge_type)

    dexp = jnp.repeat(denom + 1e-16, DK, axis=-1)

    mask = edge_type == 0
    idx_j = jnp.where(mask, j, N)
    win = jnp.full((N,), -1, jnp.int32).at[idx_j].set(
        jnp.arange(E, dtype=jnp.int32), mode='drop')
    iw = i[jnp.clip(win, 0, E - 1)]
    xw = node_inp[iw]
    wm3 = (win >= 0).astype(jnp.float32).reshape(N // BLK, BLK, 1)

    Wsp = Vw[1] @ relation_s2u[0]
    bsp = (Vb[1] @ relation_s2u[0]).reshape(1, DIM)
    alpha8 = jnp.broadcast_to(jax.nn.sigmoid(skip)[:, None], (T, DIM))

    return _run_final(aggr2, dexp, xw, wm3, nt3, node_inp,
                      Wsp, bsp, Aw, Ab, alpha8)


# speedup vs baseline: 13.6947x; 1.1149x over previous
import functools
import math

import jax
import jax.numpy as jnp
from jax import lax
from jax.experimental import pallas as pl
from jax.experimental.pallas import tpu as pltpu
from jax.experimental.pallas import tpu_sc as plsc

N = 10000
E = 320000
DIM = 128
T = 3
R = 4
NH = 8
DK = 16

BLK = 400


def _prep_body(x_ref, nt_ref, Kw_ref, Kb_ref, Qw_ref, Qb_ref, Vw_ref, Vb_ref,
               BDatt_ref, BDmsg_ref, pri_ref,
               qhat_ref, kA3_ref, vM_ref):
    x = x_ref[...]
    nt2 = nt_ref[0]

    def typed_proj(w_ref, b_ref):
        b = b_ref[...]
        outs = [jnp.dot(x, w_ref[t], preferred_element_type=jnp.float32)
                + b[t:t + 1, :] for t in range(T)]
        sel = outs[T - 1]
        for t in range(T - 2, -1, -1):
            sel = jnp.where(nt2 == t, outs[t], sel)
        return sel

    k_node = typed_proj(Kw_ref, Kb_ref)
    q_node = typed_proj(Qw_ref, Qb_ref)
    v_node = typed_proj(Vw_ref, Vb_ref)
    qhat_ref[...] = q_node * (1.0 / math.sqrt(DK))
    for r in range(R):
        kA_r = jnp.dot(k_node, BDatt_ref[r], preferred_element_type=jnp.float32)
        vM_ref[r] = jnp.dot(v_node, BDmsg_ref[r], preferred_element_type=jnp.float32)
        for ti in range(T):
            pri = pri_ref[ti, r]
            pri_sel = pri[T - 1:T, :]
            for tt in range(T - 2, -1, -1):
                pri_sel = jnp.where(nt2 == tt, pri[tt:tt + 1, :], pri_sel)
            kA3_ref[ti, r] = kA_r * pri_sel


def _run_prep(node_inp, nt3, Kw, Kb, Qw, Qb, Vw, Vb, BDatt, BDmsg, pri16):
    nb = N // BLK
    grid = (nb,)
    full = lambda *s: pl.BlockSpec(s, lambda ib: tuple(0 for _ in s))
    in_specs = [
        pl.BlockSpec((BLK, DIM), lambda ib: (ib, 0)),
        pl.BlockSpec((1, BLK, 1), lambda ib: (ib, 0, 0)),
        full(T, DIM, DIM), full(T, DIM),
        full(T, DIM, DIM), full(T, DIM),
        full(T, DIM, DIM), full(T, DIM),
        full(R, DIM, DIM), full(R, DIM, DIM),
        full(T, R, T, DIM),
    ]
    out_specs = [
        pl.BlockSpec((BLK, DIM), lambda ib: (ib, 0)),
        pl.BlockSpec((T, R, BLK, DIM), lambda ib: (0, 0, ib, 0)),
        pl.BlockSpec((R, BLK, DIM), lambda ib: (0, ib, 0)),
    ]
    out_shape = [
        jax.ShapeDtypeStruct((N, DIM), jnp.float32),
        jax.ShapeDtypeStruct((T, R, N, DIM), jnp.float32),
        jax.ShapeDtypeStruct((R, N, DIM), jnp.float32),
    ]
    return pl.pallas_call(
        _prep_body, grid=grid, in_specs=in_specs, out_specs=out_specs,
        out_shape=out_shape,
    )(node_inp, nt3, Kw, Kb, Qw, Qb, Vw, Vb, BDatt, BDmsg, pri16)


def _final_body(a_ref, dexp_ref, xw_ref, wm_ref, nt_ref, x_ref,
                Wsp_ref, bsp_ref, Aw_ref, Ab_ref, alpha_ref, out_ref):
    a = a_ref[0] + a_ref[1]
    a = a / dexp_ref[...]
    sp = jnp.dot(xw_ref[...], Wsp_ref[...], preferred_element_type=jnp.float32)
    sp = sp + bsp_ref[...]
    wm = wm_ref[0]
    nt2 = nt_ref[0]
    a = a + jnp.where(wm > 0, sp, 0.0)
    g = 0.5 * a * (1.0 + lax.erf(a * (1.0 / math.sqrt(2.0))))
    Abv = Ab_ref[...]
    outs = [jnp.dot(g, Aw_ref[t], preferred_element_type=jnp.float32)
            + Abv[t:t + 1, :] for t in range(T)]
    sel = outs[T - 1]
    alv = alpha_ref[...]
    al = alv[T - 1:T, :]
    for t in range(T - 2, -1, -1):
        m = nt2 == t
        sel = jnp.where(m, outs[t], sel)
        al = jnp.where(m, alv[t:t + 1, :], al)
    out_ref[...] = sel * al + x_ref[...] * (1.0 - al)


def _run_final(aggr2, dexp, xw, wm3, nt3, node_inp, Wsp, bsp, Aw, Ab, alpha8):
    nb = N // BLK
    full = lambda *s: pl.BlockSpec(s, lambda ib: tuple(0 for _ in s))
    in_specs = [
        pl.BlockSpec((2, BLK, DIM), lambda ib: (0, ib, 0)),
        pl.BlockSpec((BLK, DIM), lambda ib: (ib, 0)),
        pl.BlockSpec((BLK, DIM), lambda ib: (ib, 0)),
        pl.BlockSpec((1, BLK, 1), lambda ib: (ib, 0, 0)),
        pl.BlockSpec((1, BLK, 1), lambda ib: (ib, 0, 0)),
        pl.BlockSpec((BLK, DIM), lambda ib: (ib, 0)),
        full(DIM, DIM), full(1, DIM),
        full(T, DIM, DIM), full(T, DIM),
        full(T, DIM),
    ]
    return pl.pallas_call(
        _final_body, grid=(nb,), in_specs=in_specs,
        out_specs=pl.BlockSpec((BLK, DIM), lambda ib: (ib, 0)),
        out_shape=jax.ShapeDtypeStruct((N, DIM), jnp.float32),
    )(aggr2, dexp, xw, wm3, nt3, node_inp, Wsp, bsp, Aw, Ab, alpha8)


NC = 2
NS = 16
NW = NC * NS
CH = 32
CPS = 16
SUP = CH * CPS
NSUP = 20
EPT = SUP * NSUP
EPAD = NW * EPT
NACC = 10240
NPT = NACC // NS
SCAT = 0


def _edge_sc_body(qhat_hbm, kA3_hbm, vM_hbm, nt_hbm, dst_hbm, src_hbm, et_hbm,
                  aggr_out, den_out,
                  ebi, ebj, ebr, tvb, ikb, ivb,
                  q0, q1, k0, k1, vg0, vg1, vs0, vs1, w0, w1, ic0, ic1,
                  ash, dsh, gs0, gs1, ss0, ss1, ns):
    cid = lax.axis_index("c")
    sid = lax.axis_index("s")
    wid = cid * NS + sid
    ebase = wid * EPT
    f16z = jnp.zeros((16,), jnp.float32)
    i16z = jnp.zeros((16,), jnp.int32)
    iota16 = jax.lax.iota(jnp.int32, 16)
    qb = (q0, q1)
    kb = (k0, k1)
    vgb = (vg0, vg1)
    vsb = (vs0, vs1)
    wb = (w0, w1)
    icb = (ic0, ic1)
    gsem = (gs0, gs1)
    ssem = (ss0, ss1)

    def _zrow(rr, carry):
        for cc in range(DIM // 16):
            vs0[rr, pl.ds(cc * 16, 16)] = f16z
            vs1[rr, pl.ds(cc * 16, 16)] = f16z
        w0[rr, :] = f16z
        w1[rr, :] = f16z
        return carry
    lax.fori_loop(0, CH, _zrow, 0)
    for g in range(CH // 16):
        ic0[pl.ds(g * 16, 16)] = i16z
        ic1[pl.ds(g * 16, 16)] = i16z
    for kk in range(NPT // CH):
        sl = pl.ds(sid * NPT + kk * CH, CH)
        pltpu.sync_copy(vs0, ash.at[sl])
        pltpu.sync_copy(w0, dsh.at[sl])
    plsc.subcore_barrier()
    for b in range(2):
        pltpu.async_copy(vsb[b], ash.at[icb[b]], ssem[b], add=True)
        pltpu.async_copy(wb[b], dsh.at[icb[b]], ssem[b], add=True)

    def _wait_rows(b):
        pltpu.make_async_copy(qhat_hbm.at[pl.ds(0, CH)], qb[b], gsem[b]).wait()
        pltpu.make_async_copy(kA3_hbm.at[pl.ds(0, CH)], kb[b], gsem[b]).wait()
        pltpu.make_async_copy(vM_hbm.at[pl.ds(0, CH)], vgb[b], gsem[b]).wait()

    def _wait_scat(b):
        pltpu.make_async_copy(vsb[b], ash.at[pl.ds(0, CH)], ssem[b]).wait()
        pltpu.make_async_copy(wb[b], dsh.at[pl.ds(0, CH)], ssem[b]).wait()

    def _issue_rows(c, b):
        off = pl.multiple_of(c * CH, CH)
        pltpu.async_copy(qhat_hbm.at[ebi.at[pl.ds(off, CH)]], qb[b], gsem[b])
        pltpu.async_copy(kA3_hbm.at[ikb.at[pl.ds(off, CH)]], kb[b], gsem[b])
        pltpu.async_copy(vM_hbm.at[ivb.at[pl.ds(off, CH)]], vgb[b], gsem[b])

    def _issue_scat(b):
        pltpu.async_copy(vsb[b], ash.at[icb[b]], ssem[b], add=True)
        pltpu.async_copy(wb[b], dsh.at[icb[b]], ssem[b], add=True)

    def _compute(c, b):
        off = pl.multiple_of(c * CH, CH)
        for g in range(CH // 16):
            icb[b][pl.ds(g * 16, 16)] = ebi[pl.ds(off + g * 16, 16)]
        for g in range(CH // 16):
            row16 = iota16 + (g * 16)

            def _dotstep(cc, accs):
                new = []
                for h in range(NH):
                    col = jnp.full((16,), h * DK, jnp.int32) + cc
                    qv = plsc.load_gather(qb[b], [row16, col])
                    kv = plsc.load_gather(kb[b], [row16, col])
                    new.append(accs[h] + qv * kv)
                return tuple(new)
            accs = lax.fori_loop(0, DK, _dotstep,
                                 tuple(f16z for _ in range(NH)))
            for h in range(NH):
                plsc.store_scatter(wb[b],
                                   [row16, jnp.full((16,), h, jnp.int32)],
                                   jnp.exp(accs[h]))

        def _scale(e, carry):
            for h in range(NH):
                wv = plsc.load_gather(
                    wb[b], [jnp.full((16,), e, jnp.int32),
                            jnp.full((16,), h, jnp.int32)])
                vsb[b][e, pl.ds(h * DK, DK)] = (
                    vgb[b][e, pl.ds(h * DK, DK)] * wv)
            return carry
        lax.fori_loop(0, CH, _scale, 0)

    def _super(S, carry):
        sbase = pl.multiple_of(ebase + S * SUP, SUP)
        h0 = pltpu.async_copy(dst_hbm.at[pl.ds(sbase, SUP)], ebi, ns)
        h1 = pltpu.async_copy(src_hbm.at[pl.ds(sbase, SUP)], ebj, ns)
        h2 = pltpu.async_copy(et_hbm.at[pl.ds(sbase, SUP)], ebr, ns)
        h0.wait()
        h1.wait()
        h2.wait()
        hs = [pltpu.async_copy(nt_hbm.at[ebi.at[pl.ds(p * 128, 128)]],
                               tvb.at[pl.ds(p * 128, 128)], ns)
              for p in range(SUP // 128)]
        for h in hs:
            h.wait()

        def _idx(g, carry2):
            j16 = ebj[pl.ds(g * 16, 16)]
            r16 = ebr[pl.ds(g * 16, 16)]
            t16 = tvb[pl.ds(g * 16, 16)]
            ikb[pl.ds(g * 16, 16)] = (t16 * R + r16) * N + j16
            ivb[pl.ds(g * 16, 16)] = r16 * N + j16
            return carry2
        lax.fori_loop(0, SUP // 16, _idx, 0)

        _issue_rows(0, 0)
        _issue_rows(1, 1)

        def _body2(kk, carry2):
            c0 = 2 * kk
            for b in range(2):
                cc = c0 + b
                _wait_scat(b)
                _wait_rows(b)
                _compute(cc, b)
                _issue_scat(b)
                _issue_rows(cc + 2, b)
            return carry2
        lax.fori_loop(0, CPS // 2 - 1, _body2, 0)
        for b in range(2):
            _wait_scat(b)
            _wait_rows(b)
            _compute(CPS - 2 + b, b)
            _issue_scat(b)
        return carry

    lax.fori_loop(0, NSUP, _super, 0)

    for b in range(2):
        _wait_scat(b)
    plsc.subcore_barrier()
    asl = pl.ds(sid * NPT, NPT)
    pltpu.sync_copy(ash.at[asl], aggr_out.at[cid, asl])
    pltpu.sync_copy(dsh.at[asl], den_out.at[cid, asl])


def _edge_phase(qhat, kA3f, vMf, nt, dst, src, et):
    mesh = plsc.VectorSubcoreMesh(core_axis_name="c", subcore_axis_name="s")
    run = pl.kernel(
        _edge_sc_body, mesh=mesh,
        compiler_params=pltpu.CompilerParams(needs_layout_passes=False,
                                             use_tc_tiling_on_sc=False),
        out_type=[
            jax.ShapeDtypeStruct((NC, NACC, DIM), jnp.float32),
            jax.ShapeDtypeStruct((NC, NACC, 16), jnp.float32),
        ],
        scratch_types=[
            pltpu.VMEM((SUP,), jnp.int32),
            pltpu.VMEM((SUP,), jnp.int32),
            pltpu.VMEM((SUP,), jnp.int32),
            pltpu.VMEM((SUP,), jnp.int32),
            pltpu.VMEM((SUP,), jnp.int32),
            pltpu.VMEM((SUP,), jnp.int32),
            pltpu.VMEM((CH, DIM), jnp.float32),
            pltpu.VMEM((CH, DIM), jnp.float32),
            pltpu.VMEM((CH, DIM), jnp.float32),
            pltpu.VMEM((CH, DIM), jnp.float32),
            pltpu.VMEM((CH, DIM), jnp.float32),
            pltpu.VMEM((CH, DIM), jnp.float32),
            pltpu.VMEM((CH, DIM), jnp.float32),
            pltpu.VMEM((CH, DIM), jnp.float32),
            pltpu.VMEM((CH, 16), jnp.float32),
            pltpu.VMEM((CH, 16), jnp.float32),
            pltpu.VMEM((CH,), jnp.int32),
            pltpu.VMEM((CH,), jnp.int32),
            pltpu.VMEM_SHARED((NACC, DIM), jnp.float32),
            pltpu.VMEM_SHARED((NACC, 16), jnp.float32),
            pltpu.SemaphoreType.DMA,
            pltpu.SemaphoreType.DMA,
            pltpu.SemaphoreType.DMA,
            pltpu.SemaphoreType.DMA,
            pltpu.SemaphoreType.DMA,
        ],
    )
    aggr2, den2 = run(qhat, kA3f, vMf, nt, dst, src, et)
    return aggr2[:, :N], den2[:, :N, :NH].sum(0)


def kernel(node_inp, node_type, edge_index, edge_type, node_position,
           Kw, Kb, Qw, Qb, Vw, Vb, Aw, Ab,
           relation_pri, relation_att, relation_msg, relation_s2u, skip):
    nt3 = node_type.reshape(N // BLK, BLK, 1)
    def blockdiag2(rel):
        out = jnp.zeros((R, DIM, DIM), jnp.float32)
        for h in range(NH):
            out = out.at[:, h * DK:(h + 1) * DK, h * DK:(h + 1) * DK].set(rel[:, h])
        return out
    BDatt = blockdiag2(relation_att)
    BDmsg = blockdiag2(relation_msg)
    pri16 = jnp.repeat(relation_pri, DK, axis=-1)

    qhat, kA3, vM = _run_prep(node_inp, nt3, Kw, Kb, Qw, Qb, Vw, Vb,
                              BDatt, BDmsg, pri16)
    kA3f = kA3.reshape(T * R * N, DIM)
    vMf = vM.reshape(R * N, DIM)

    j = edge_index[0]
    i = edge_index[1]
    pad = EPAD - E
    i_p = jnp.concatenate([i, jnp.full((pad,), N, jnp.int32)])
    j_p = jnp.concatenate([j, jnp.zeros((pad,), jnp.int32)])
    et_p = jnp.concatenate([edge_type, jnp.zeros((pad,), jnp.int32)])
    qhat_p = jnp.pad(qhat, ((0, NACC - N), (0, 0)))
    nt_p = jnp.pad(node_type, (0, NACC - N))
    aggr2, denom = _edge_phase(qhat_p, kA3f, vMf, nt_p, i_p, j_p, et_p)

    dexp = jnp.repeat(denom + 1e-16, DK, axis=-1)

    mask = edge_type == 0
    idx_j = jnp.where(mask, j, N)
    win = jnp.full((N,), -1, jnp.int32).at[idx_j].set(
        jnp.arange(E, dtype=jnp.int32), mode='drop')
    iw = i[jnp.clip(win, 0, E - 1)]
    xw = node_inp[iw]
    wm3 = (win >= 0).astype(jnp.float32).reshape(N // BLK, BLK, 1)

    Wsp = Vw[1] @ relation_s2u[0]
    bsp = (Vb[1] @ relation_s2u[0]).reshape(1, DIM)
    alpha8 = jnp.broadcast_to(jax.nn.sigmoid(skip)[:, None], (T, DIM))

    return _run_final(aggr2, dexp, xw, wm3, nt3, node_inp,
                      Wsp, bsp, Aw, Ab, alpha8)
